# Initial kernel scaffold; baseline (speedup 1.0000x reference)
import jax
import jax.numpy as jnp
from jax.experimental import pallas as pl


def kernel(x, edge_index, batch, W_gc, b_gc, W_mu, b_mu, W_lv, b_lv, W_dec, b_dec):
    raise NotImplementedError("write your pallas kernel here")


# speedup vs baseline: 32.6354x; 32.6354x over previous
import functools

import jax
import jax.numpy as jnp
from jax import lax
from jax.experimental import pallas as pl
from jax.experimental.pallas import tpu as pltpu
from jax.experimental.pallas import tpu_sc as plsc

NC = 2
NS = 16
NW = NC * NS
CHUNK = 128


def _cdiv(a, b):
    return (a + b - 1) // b


def _sc_degree(col3, np_rows, per_tile, k_chunks):
    mesh = plsc.VectorSubcoreMesh(core_axis_name="c", subcore_axis_name="s")

    @functools.partial(
        pl.kernel,
        out_type=jax.ShapeDtypeStruct((NC * np_rows,), jnp.float32),
        mesh=mesh,
        compiler_params=pltpu.CompilerParams(use_tc_tiling_on_sc=False),
        scratch_types=[
            pltpu.VMEM((k_chunks, CHUNK), jnp.int32),
            pltpu.VMEM((CHUNK,), jnp.float32),
            pltpu.VMEM((per_tile,), jnp.float32),
            pltpu.VMEM_SHARED((np_rows,), jnp.float32),
        ],
    )
    def k(col_hbm, out_hbm, colv, onesv, zv, deg_sh):
        c = lax.axis_index("c")
        s = lax.axis_index("s")
        wid = c * NS + s
        ones16 = jnp.ones((16,), jnp.float32)
        zeros16 = jnp.zeros((16,), jnp.float32)
        for i in range(CHUNK // 16):
            onesv[pl.ds(i * 16, 16)] = ones16

        def zb(i, carry):
            zv[pl.ds(i * 16, 16)] = zeros16
            return carry

        lax.fori_loop(0, per_tile // 16, zb, 0)
        pltpu.sync_copy(zv, deg_sh.at[pl.ds(s * per_tile, per_tile)])
        plsc.subcore_barrier()
        pltpu.sync_copy(col_hbm.at[wid], colv)

        def body(j, carry):
            pltpu.sync_copy(onesv, deg_sh.at[colv.at[j]], add=True)
            return carry

        lax.fori_loop(0, k_chunks, body, 0)
        plsc.subcore_barrier()
        pltpu.sync_copy(deg_sh.at[pl.ds(s * per_tile, per_tile)], zv)
        pltpu.sync_copy(zv, out_hbm.at[pl.ds(c * np_rows + s * per_tile,
                                             per_tile)])

    return k(col3).reshape(NC, np_rows)


def _sc_scatter(y, row3, col3, np_rows, per_tile, k_chunks, h):
    mesh = plsc.VectorSubcoreMesh(core_axis_name="c", subcore_axis_name="s")

    @functools.partial(
        pl.kernel,
        out_type=jax.ShapeDtypeStruct((NC, np_rows, h), jnp.float32),
        mesh=mesh,
        compiler_params=pltpu.CompilerParams(use_tc_tiling_on_sc=False),
        scratch_types=[
            pltpu.VMEM((k_chunks, CHUNK), jnp.int32),
            pltpu.VMEM((k_chunks, CHUNK), jnp.int32),
            pltpu.VMEM((CHUNK, h), jnp.float32),
            pltpu.VMEM((per_tile, h), jnp.float32),
            pltpu.VMEM_SHARED((np_rows, h), jnp.float32),
            pltpu.SemaphoreType.DMA,
        ],
    )
    def k(y_hbm, row_hbm, col_hbm, out_hbm, rowv, colv, rbuf, zbuf, acc_sh, sem):
        c = lax.axis_index("c")
        s = lax.axis_index("s")
        wid = c * NS + s
        zeros16 = jnp.zeros((16,), jnp.float32)

        def zb(i, carry):
            for t in range(h // 16):
                zbuf[i, pl.ds(t * 16, 16)] = zeros16
            return carry

        lax.fori_loop(0, per_tile, zb, 0)
        pltpu.sync_copy(zbuf, acc_sh.at[pl.ds(s * per_tile, per_tile)])
        plsc.subcore_barrier()
        pltpu.sync_copy(row_hbm.at[wid], rowv)
        pltpu.sync_copy(col_hbm.at[wid], colv)

        def body(j, carry):
            pltpu.async_copy(y_hbm.at[rowv.at[j]], rbuf, sem).wait()
            pltpu.sync_copy(rbuf, acc_sh.at[colv.at[j]], add=True)
            return carry

        lax.fori_loop(0, k_chunks, body, 0)
        plsc.subcore_barrier()
        pltpu.sync_copy(acc_sh.at[pl.ds(s * per_tile, per_tile)], zbuf)
        pltpu.sync_copy(zbuf, out_hbm.at[c, pl.ds(s * per_tile, per_tile)])

    return k(y, row3, col3)


def _prescale_body(x_ref, w_ref, degp_ref, y_ref, dis_ref):
    xw = jnp.dot(x_ref[...], w_ref[...], preferred_element_type=jnp.float32)
    deg = degp_ref[0] + degp_ref[1] + 1.0
    dis = lax.rsqrt(deg)
    dis_ref[...] = dis
    y_ref[...] = xw * dis


def _tc_prescale(x, w_gc, degp3, bn):
    n, d = x.shape
    h = w_gc.shape[1]
    g = n // bn
    return pl.pallas_call(
        _prescale_body,
        grid=(g,),
        in_specs=[
            pl.BlockSpec((bn, d), lambda i: (i, 0)),
            pl.BlockSpec((d, h), lambda i: (0, 0)),
            pl.BlockSpec((NC, bn, 1), lambda i: (0, i, 0)),
        ],
        out_specs=[
            pl.BlockSpec((bn, h), lambda i: (i, 0)),
            pl.BlockSpec((bn, 1), lambda i: (i, 0)),
        ],
        out_shape=[
            jax.ShapeDtypeStruct((n, h), jnp.float32),
            jax.ShapeDtypeStruct((n, 1), jnp.float32),
        ],
    )(x, w_gc, degp3)


def _make_heads_body(bn, bq, g):
    def body(accp_ref, y_ref, dis_ref, batch_ref, bgc_ref, eps_ref,
             wmu_ref, bmu_ref, wlv_ref, blv_ref, wdec_ref, bdec_ref,
             adj_ref, mu_ref, lv_ref, sums, cnt):
        i = pl.program_id(0)

        @pl.when(i == 0)
        def _():
            sums[...] = jnp.zeros_like(sums)
            cnt[...] = jnp.zeros_like(cnt)

        acc = accp_ref[0] + accp_ref[1]
        hrow = jnp.maximum(
            dis_ref[...] * (acc + y_ref[...]) + bgc_ref[...], 0.0)
        bt = batch_ref[0]
        onehot_t = (lax.broadcasted_iota(jnp.int32, (bq, bn), 0) == bt
                    ).astype(jnp.float32)
        sums[...] += jnp.dot(onehot_t, hrow, preferred_element_type=jnp.float32)
        cnt[...] += jnp.dot(onehot_t, jnp.ones((bn, 1), jnp.float32),
                            preferred_element_type=jnp.float32)

        @pl.when(i == g - 1)
        def _():
            hm = sums[...] / jnp.maximum(cnt[...], 1.0)
            mu = jnp.dot(hm, wmu_ref[...],
                         preferred_element_type=jnp.float32) + bmu_ref[...]
            lv = jnp.dot(hm, wlv_ref[...],
                         preferred_element_type=jnp.float32) + blv_ref[...]
            std = jnp.exp(0.5 * lv)
            z = mu + eps_ref[...] * std
            logits = jnp.dot(z, wdec_ref[...],
                             preferred_element_type=jnp.float32) + bdec_ref[...]
            adj_ref[...] = (logits > 0.0).astype(jnp.float32)
            mu_ref[...] = mu
            lv_ref[...] = lv

    return body


def _tc_heads(accp, y, dis, batch3, b_gc, eps, w_mu, b_mu, w_lv, b_lv,
              w_dec, b_dec, bn, bq):
    n, h = y.shape
    z = w_mu.shape[1]
    a = w_dec.shape[1]
    g = n // bn
    full = lambda shape: pl.BlockSpec(shape, lambda i: tuple(0 for _ in shape))
    return pl.pallas_call(
        _make_heads_body(bn, bq, g),
        grid=(g,),
        in_specs=[
            pl.BlockSpec((NC, bn, h), lambda i: (0, i, 0)),
            pl.BlockSpec((bn, h), lambda i: (i, 0)),
            pl.BlockSpec((bn, 1), lambda i: (i, 0)),
            pl.BlockSpec((1, 1, bn), lambda i: (i, 0, 0)),
            full((h,)),
            full((bq, z)),
            full((h, z)),
            full((z,)),
            full((h, z)),
            full((z,)),
            full((z, a)),
            full((a,)),
        ],
        out_specs=[
            full((bq, a)),
            full((bq, z)),
            full((bq, z)),
        ],
        out_shape=[
            jax.ShapeDtypeStruct((bq, a), jnp.float32),
            jax.ShapeDtypeStruct((bq, z), jnp.float32),
            jax.ShapeDtypeStruct((bq, z), jnp.float32),
        ],
        scratch_shapes=[
            pltpu.VMEM((bq, h), jnp.float32),
            pltpu.VMEM((bq, 1), jnp.float32),
        ],
    )(accp, y, dis, batch3, b_gc, eps, w_mu, b_mu, w_lv, b_lv, w_dec, b_dec)


def kernel(x, edge_index, batch, W_gc, b_gc, W_mu, b_mu, W_lv, b_lv,
           W_dec, b_dec):
    n, d = x.shape
    h = W_gc.shape[1]
    z = W_mu.shape[1]
    e = edge_index.shape[1]
    bq = 512
    bn = 1000

    k_chunks = _cdiv(e, NW * CHUNK)
    e_pad = NW * k_chunks * CHUNK
    pad = e_pad - e
    row = edge_index[0].astype(jnp.int32)
    col = edge_index[1].astype(jnp.int32)
    if pad:
        row = jnp.concatenate([row, jnp.zeros((pad,), jnp.int32)])
        col = jnp.concatenate([col, jnp.full((pad,), n, jnp.int32)])
    row3 = row.reshape(NW, k_chunks, CHUNK)
    col3 = col.reshape(NW, k_chunks, CHUNK)

    per_tile = _cdiv(n + 1, NS * 8) * 8
    np_rows = NS * per_tile

    degp = _sc_degree(col3, np_rows, per_tile, k_chunks)
    degp3 = degp[:, :, None]
    y, dis = _tc_prescale(x, W_gc, degp3, bn)
    accp = _sc_scatter(y, row3, col3, np_rows, per_tile, k_chunks, h)

    eps = jax.random.normal(jax.random.key(42), (bq, z), jnp.float32)
    batch3 = batch.astype(jnp.int32).reshape(n // bn, 1, bn)
    adj_flat, mu, logvar = _tc_heads(
        accp, y, dis, batch3, b_gc, eps, W_mu, b_mu, W_lv, b_lv,
        W_dec, b_dec, bn, bq)
    adj = adj_flat.reshape(-1, 10, 10)
    return adj, mu, logvar
